# initial kernel scaffold (unmeasured)
import jax
import jax.numpy as jnp
from jax import lax
from jax.experimental import pallas as pl
from jax.experimental.pallas import tpu as pltpu

T = 2048
D = 4096
V_SHARD = 8192
BLK_T = T // 4
VT = 512
NEG = -1e30


def _partial_kernel(x_blk, w, labels_blk):
    n_tiles = V_SHARD // VT

    def body(x_ref, w_ref, lab_ref, m_ref, s_ref, g_ref):
        j = pl.program_id(0)
        my_y = lax.axis_index("y")
        logits = jnp.dot(
            x_ref[...], w_ref[...], preferred_element_type=jnp.float32
        )

        @pl.when(j == 0)
        def _():
            m_ref[...] = jnp.full((BLK_T, 1), NEG, jnp.float32)
            s_ref[...] = jnp.zeros((BLK_T, 1), jnp.float32)
            g_ref[...] = jnp.zeros((BLK_T, 1), jnp.float32)

        m_prev = m_ref[...]
        tile_m = jnp.max(logits, axis=1, keepdims=True)
        m_new = jnp.maximum(m_prev, tile_m)
        s_ref[...] = s_ref[...] * jnp.exp(m_prev - m_new) + jnp.sum(
            jnp.exp(logits - m_new), axis=1, keepdims=True
        )
        m_ref[...] = m_new

        base = my_y * V_SHARD + j * VT
        rel = lab_ref[...] - base
        col = lax.broadcasted_iota(jnp.int32, logits.shape, 1)
        g_ref[...] = g_ref[...] + jnp.sum(
            jnp.where(col == rel, logits, 0.0), axis=1, keepdims=True
        )

    return pl.pallas_call(
        body,
        grid=(n_tiles,),
        in_specs=[
            pl.BlockSpec((BLK_T, D), lambda j: (0, 0)),
            pl.BlockSpec((D, VT), lambda j: (0, j)),
            pl.BlockSpec((BLK_T, 1), lambda j: (0, 0)),
        ],
        out_specs=[pl.BlockSpec((BLK_T, 1), lambda j: (0, 0))] * 3,
        out_shape=[jax.ShapeDtypeStruct((BLK_T, 1), jnp.float32)] * 3,
    )(x_blk, w, labels_blk)


def _allreduce_kernel(packed):

    def body(p_ref, out_ref, acc_ref, buf_ref, send_sems, recv_sems):
        my_x = lax.axis_index("x")
        my_y = lax.axis_index("y")
        my_z = lax.axis_index("z")
        acc_ref[...] = p_ref[...]
        partners = [
            (my_x, my_y, 1 - my_z),
            (1 - my_x, my_y, my_z),
            (my_x, 1 - my_y, my_z),
        ]
        for r in range(3):
            rdma = pltpu.make_async_remote_copy(
                src_ref=acc_ref,
                dst_ref=buf_ref.at[r],
                send_sem=send_sems.at[r],
                recv_sem=recv_sems.at[r],
                device_id=partners[r],
                device_id_type=pl.DeviceIdType.MESH,
            )
            rdma.start()
            rdma.wait()
            m = acc_ref[0:1, :]
            s = acc_ref[1:2, :]
            g = acc_ref[2:3, :]
            mo = buf_ref[r, 0:1, :]
            so = buf_ref[r, 1:2, :]
            go = buf_ref[r, 2:3, :]
            mn = jnp.maximum(m, mo)
            sn = s * jnp.exp(m - mn) + so * jnp.exp(mo - mn)
            acc_ref[0:1, :] = mn
            acc_ref[1:2, :] = sn
            acc_ref[2:3, :] = g + go
        out_ref[...] = (
            acc_ref[0:1, :] + jnp.log(acc_ref[1:2, :]) - acc_ref[2:3, :]
        )

    return pl.pallas_call(
        body,
        out_shape=jax.ShapeDtypeStruct((1, T), jnp.float32),
        in_specs=[pl.BlockSpec(memory_space=pltpu.VMEM)],
        out_specs=pl.BlockSpec(memory_space=pltpu.VMEM),
        scratch_shapes=[
            pltpu.VMEM((3, T), jnp.float32),
            pltpu.VMEM((3, 3, T), jnp.float32),
            pltpu.SemaphoreType.DMA((3,)),
            pltpu.SemaphoreType.DMA((3,)),
        ],
        compiler_params=pltpu.CompilerParams(collective_id=0),
    )(packed)


def kernel(x, W, labels):
    my_x = lax.axis_index("x")
    my_z = lax.axis_index("z")
    b = my_x * 2 + my_z
    row0 = b * BLK_T
    x_blk = lax.dynamic_slice(x, (row0, 0), (BLK_T, D))
    lab_blk = lax.dynamic_slice(labels, (row0,), (BLK_T,)).reshape(BLK_T, 1)

    m, s, g = _partial_kernel(x_blk, W, lab_blk)

    M = lax.dynamic_update_slice(
        jnp.full((T,), NEG, jnp.float32), m.reshape(BLK_T), (row0,)
    )
    S = lax.dynamic_update_slice(
        jnp.zeros((T,), jnp.float32), s.reshape(BLK_T), (row0,)
    )
    G = lax.dynamic_update_slice(
        jnp.zeros((T,), jnp.float32), g.reshape(BLK_T), (row0,)
    )
    packed = jnp.stack([M, S, G])

    nll = _allreduce_kernel(packed)
    return nll.reshape(T)


# baseline (device time: 77004 ns/iter reference)
import jax
import jax.numpy as jnp
from jax import lax
from jax.experimental import pallas as pl
from jax.experimental.pallas import tpu as pltpu

T = 2048
D = 4096
V_SHARD = 8192
BLK_T = T // 4
VT = 512
NEG = -1e30


def _partial_kernel(x_blk, w, labels_blk):
    n_tiles = V_SHARD // VT

    def body(x_ref, w_ref, lab_ref, m_ref, s_ref, g_ref):
        j = pl.program_id(0)
        my_y = lax.axis_index("y")
        logits = jnp.dot(
            x_ref[...], w_ref[...], preferred_element_type=jnp.float32
        )

        @pl.when(j == 0)
        def _():
            m_ref[...] = jnp.full((BLK_T, 1), NEG, jnp.float32)
            s_ref[...] = jnp.zeros((BLK_T, 1), jnp.float32)
            g_ref[...] = jnp.zeros((BLK_T, 1), jnp.float32)

        m_prev = m_ref[...]
        tile_m = jnp.max(logits, axis=1, keepdims=True)
        m_new = jnp.maximum(m_prev, tile_m)
        s_ref[...] = s_ref[...] * jnp.exp(m_prev - m_new) + jnp.sum(
            jnp.exp(logits - m_new), axis=1, keepdims=True
        )
        m_ref[...] = m_new

        base = my_y * V_SHARD + j * VT
        rel = lab_ref[...] - base
        col = lax.broadcasted_iota(jnp.int32, logits.shape, 1)
        g_ref[...] = g_ref[...] + jnp.sum(
            jnp.where(col == rel, logits, 0.0), axis=1, keepdims=True
        )

    return pl.pallas_call(
        body,
        grid=(n_tiles,),
        in_specs=[
            pl.BlockSpec((BLK_T, D), lambda j: (0, 0)),
            pl.BlockSpec((D, VT), lambda j: (0, j)),
            pl.BlockSpec((BLK_T, 1), lambda j: (0, 0)),
        ],
        out_specs=[pl.BlockSpec((BLK_T, 1), lambda j: (0, 0))] * 3,
        out_shape=[jax.ShapeDtypeStruct((BLK_T, 1), jnp.float32)] * 3,
    )(x_blk, w, labels_blk)


def _allreduce_kernel(packed):

    def body(p_ref, out_ref, acc_ref, buf_ref, send_sems, recv_sems):
        my_x = lax.axis_index("x")
        my_y = lax.axis_index("y")
        my_z = lax.axis_index("z")
        acc_ref[...] = p_ref[...]
        partners = [
            (my_x, my_y, 1 - my_z),
            (1 - my_x, my_y, my_z),
            (my_x, 1 - my_y, my_z),
        ]
        for r in range(3):
            rdma = pltpu.make_async_remote_copy(
                src_ref=acc_ref,
                dst_ref=buf_ref.at[r],
                send_sem=send_sems.at[r],
                recv_sem=recv_sems.at[r],
                device_id=partners[r],
                device_id_type=pl.DeviceIdType.MESH,
            )
            rdma.start()
            rdma.wait()
            m = acc_ref[0:1, :]
            s = acc_ref[1:2, :]
            g = acc_ref[2:3, :]
            mo = buf_ref[r, 0:1, :]
            so = buf_ref[r, 1:2, :]
            go = buf_ref[r, 2:3, :]
            mn = jnp.maximum(m, mo)
            sn = s * jnp.exp(m - mn) + so * jnp.exp(mo - mn)
            acc_ref[0:1, :] = mn
            acc_ref[1:2, :] = sn
            acc_ref[2:3, :] = g + go
        out_ref[...] = (
            acc_ref[0:1, :] + jnp.log(acc_ref[1:2, :]) - acc_ref[2:3, :]
        )

    return pl.pallas_call(
        body,
        out_shape=jax.ShapeDtypeStruct((1, T), jnp.float32),
        in_specs=[pl.BlockSpec(memory_space=pltpu.VMEM)],
        out_specs=pl.BlockSpec(memory_space=pltpu.VMEM),
        scratch_shapes=[
            pltpu.VMEM((3, T), jnp.float32),
            pltpu.VMEM((3, 3, T), jnp.float32),
            pltpu.SemaphoreType.DMA((3,)),
            pltpu.SemaphoreType.DMA((3,)),
        ],
    )(packed)


def kernel(x, W, labels):
    my_x = lax.axis_index("x")
    my_z = lax.axis_index("z")
    b = my_x * 2 + my_z
    row0 = b * BLK_T
    x_blk = lax.dynamic_slice(x, (row0, 0), (BLK_T, D))
    lab_blk = lax.dynamic_slice(labels, (row0,), (BLK_T,)).reshape(BLK_T, 1)

    m, s, g = _partial_kernel(x_blk, W, lab_blk)

    M = lax.dynamic_update_slice(
        jnp.full((T,), NEG, jnp.float32), m.reshape(BLK_T), (row0,)
    )
    S = lax.dynamic_update_slice(
        jnp.zeros((T,), jnp.float32), s.reshape(BLK_T), (row0,)
    )
    G = lax.dynamic_update_slice(
        jnp.zeros((T,), jnp.float32), g.reshape(BLK_T), (row0,)
    )
    packed = jnp.stack([M, S, G])

    nll = _allreduce_kernel(packed)
    return nll.reshape(T)
